# baseline (speedup 1.0000x reference)
import jax
import jax.numpy as jnp
from jax.experimental import pallas as pl
from jax.experimental.pallas import tpu as pltpu

MAXLEN = 8192
OUTPUT_DIM = 1024
N_CHUNKS = 8
CHUNK = MAXLEN // N_CHUNKS


def _dma_body(tab_ref, out_ref, sems):
    copies = [
        pltpu.make_async_copy(
            tab_ref.at[pl.ds(i * CHUNK, CHUNK), :],
            out_ref.at[0, pl.ds(i * CHUNK, CHUNK), :],
            sems.at[i],
        )
        for i in range(N_CHUNKS)
    ]
    for c in copies:
        c.start()
    for c in copies:
        c.wait()


def kernel(inputs, table):
    del inputs
    out = pl.pallas_call(
        _dma_body,
        in_specs=[pl.BlockSpec(memory_space=pl.ANY)],
        out_specs=pl.BlockSpec(memory_space=pl.ANY),
        out_shape=jax.ShapeDtypeStruct((1, MAXLEN, OUTPUT_DIM), table.dtype),
        scratch_shapes=[pltpu.SemaphoreType.DMA((N_CHUNKS,))],
    )(table)
    return out


# speedup vs baseline: 45.1503x; 45.1503x over previous
import jax
import jax.numpy as jnp
from jax.experimental import pallas as pl

MAXLEN = 8192
OUTPUT_DIM = 1024
BLOCK_ROWS = 1024


def _copy_body(tab_ref, out_ref):
    out_ref[0] = tab_ref[...]


def kernel(inputs, table):
    del inputs
    grid = (MAXLEN // BLOCK_ROWS,)
    out = pl.pallas_call(
        _copy_body,
        grid=grid,
        in_specs=[pl.BlockSpec((BLOCK_ROWS, OUTPUT_DIM), lambda i: (i, 0))],
        out_specs=pl.BlockSpec((1, BLOCK_ROWS, OUTPUT_DIM), lambda i: (0, i, 0)),
        out_shape=jax.ShapeDtypeStruct((1, MAXLEN, OUTPUT_DIM), table.dtype),
    )(table)
    return out


# speedup vs baseline: 48.8632x; 1.0822x over previous
import jax
import jax.numpy as jnp
from jax.experimental import pallas as pl

MAXLEN = 8192
OUTPUT_DIM = 1024
BLOCK_ROWS = 2048


def _copy_body(tab_ref, out_ref):
    out_ref[0] = tab_ref[...]


def kernel(inputs, table):
    del inputs
    grid = (MAXLEN // BLOCK_ROWS,)
    out = pl.pallas_call(
        _copy_body,
        grid=grid,
        in_specs=[pl.BlockSpec((BLOCK_ROWS, OUTPUT_DIM), lambda i: (i, 0))],
        out_specs=pl.BlockSpec((1, BLOCK_ROWS, OUTPUT_DIM), lambda i: (0, i, 0)),
        out_shape=jax.ShapeDtypeStruct((1, MAXLEN, OUTPUT_DIM), table.dtype),
    )(table)
    return out
